# baseline (speedup 1.0000x reference)
import functools

import jax
import jax.numpy as jnp
from jax import lax
from jax.experimental import pallas as pl
from jax.experimental.pallas import tpu as pltpu
from jax.experimental.pallas import tpu_sc as plsc

NC = 2
NS = 16
NW = NC * NS
RAYS = 65536
SAMPLES = 256
RPW = RAYS // NW
CR = 32
CHUNKS = RPW // CR
NPAIRS = CHUNKS // 2
CI = CR * SAMPLES
KROWS = CI // 128
DLEN = 128
NDESC = CI // DLEN
ROWS_PER_WORKER = RPW * SAMPLES // 128

_FLIP_MASK = 255 << 16


def _sc_sums(vol_flat, idxs2d, step_flat):
    mesh = plsc.VectorSubcoreMesh(core_axis_name="c", subcore_axis_name="s")

    @functools.partial(
        pl.kernel,
        mesh=mesh,
        out_type=jax.ShapeDtypeStruct((RAYS,), jnp.float32),
        scratch_types=[
            pltpu.VMEM((CI,), jnp.int32),
            pltpu.VMEM((CI,), jnp.int32),
            pltpu.VMEM((CI,), jnp.float32),
            pltpu.VMEM((CI,), jnp.float32),
            pltpu.VMEM((CI,), jnp.float32),
            pltpu.VMEM((CI,), jnp.float32),
            pltpu.VMEM((RPW,), jnp.float32),
            pltpu.SemaphoreType.DMA,
            pltpu.SemaphoreType.DMA,
            pltpu.SemaphoreType.DMA,
        ],
    )
    def k(vol_hbm, idx_hbm, step_hbm, out_hbm,
          tidx0, tidx1, gbuf0, gbuf1, step0, step1, outbuf,
          sem_in, sem_g0, sem_g1):
        wid = lax.axis_index("s") * NC + lax.axis_index("c")
        ray0 = wid * RPW
        row0 = wid * ROWS_PER_WORKER
        lanes = lax.iota(jnp.int32, 16)
        perms = [lanes ^ st for st in (1, 2, 4, 8)]

        def start_in(c, tidx, stepb):
            pltpu.async_copy(
                idx_hbm.at[pl.ds((ray0 + c * CR) * SAMPLES, CI)], tidx, sem_in)
            pltpu.async_copy(
                step_hbm.at[pl.ds((ray0 + c * CR) * SAMPLES, CI)], stepb,
                sem_in)

        def wait_in(c, tidx, stepb):
            pltpu.make_async_copy(
                idx_hbm.at[pl.ds((ray0 + c * CR) * SAMPLES, CI)], tidx,
                sem_in).wait()
            pltpu.make_async_copy(
                step_hbm.at[pl.ds((ray0 + c * CR) * SAMPLES, CI)], stepb,
                sem_in).wait()

        def transform(tidx):
            def trow(rw, _):
                for j in range(8):
                    off = rw * 128 + j * 16
                    tidx[pl.ds(off, 16)] = tidx[pl.ds(off, 16)] ^ _FLIP_MASK
                return 0

            lax.fori_loop(0, KROWS, trow, 0)

        def fire_g(tidx, gbuf, sem):
            def fire(jr, _):
                off = jr * DLEN
                pltpu.async_copy(
                    vol_hbm.at[tidx.at[pl.ds(off, DLEN)]],
                    gbuf.at[pl.ds(off, DLEN)], sem)
                return 0

            lax.fori_loop(0, NDESC, fire, 0)

        def drain_g(tidx, gbuf, sem):
            def drain(jr, _):
                off = jr * DLEN
                pltpu.make_async_copy(
                    vol_hbm.at[tidx.at[pl.ds(off, DLEN)]],
                    gbuf.at[pl.ds(off, DLEN)], sem).wait()
                return 0

            lax.fori_loop(0, NDESC, drain, 0)

        def compute(c, gbuf, stepb):
            for grp in range(CR // 16):
                def ray_body(r, outvec, grp=grp):
                    rr = grp * 16 + r
                    acc = jnp.zeros((16,), jnp.float32)
                    for j in range(16):
                        g = gbuf[pl.ds(rr * SAMPLES + j * 16, 16)]
                        s = stepb[pl.ds(rr * SAMPLES + j * 16, 16)]
                        acc = acc + g * s
                    for p in perms:
                        acc = acc + acc.at[p].get(mode="promise_in_bounds")
                    return jnp.where(lanes == r, acc, outvec)

                outvec = lax.fori_loop(
                    0, 16, ray_body, jnp.zeros((16,), jnp.float32))
                outbuf[pl.ds(c * CR + grp * 16, 16)] = outvec

        start_in(0, tidx0, step0)

        def pair_body(p, _):
            c0 = 2 * p
            c1 = c0 + 1
            wait_in(c0, tidx0, step0)
            transform(tidx0)
            fire_g(tidx0, gbuf0, sem_g0)

            @pl.when(p > 0)
            def _():
                drain_g(tidx1, gbuf1, sem_g1)
                compute(c0 - 1, gbuf1, step1)

            start_in(c1, tidx1, step1)
            wait_in(c1, tidx1, step1)
            transform(tidx1)
            fire_g(tidx1, gbuf1, sem_g1)
            drain_g(tidx0, gbuf0, sem_g0)
            compute(c0, gbuf0, step0)

            @pl.when(p < NPAIRS - 1)
            def _():
                start_in(c0 + 2, tidx0, step0)

            return 0

        lax.fori_loop(0, NPAIRS, pair_body, 0)
        drain_g(tidx1, gbuf1, sem_g1)
        compute(CHUNKS - 1, gbuf1, step1)
        pltpu.sync_copy(outbuf, out_hbm.at[pl.ds(ray0, RPW)])

    return k(vol_flat, idxs2d, step_flat)


def _tc_finish(sums2d, t3, s3):
    def body(sum_ref, t_ref, s_ref, o_ref):
        d = t_ref[...] - s_ref[...] + jnp.float32(1e-8)
        rl = jnp.sqrt(jnp.sum(d * d, axis=0))
        o_ref[...] = sum_ref[...] * rl

    return pl.pallas_call(
        body,
        out_shape=jax.ShapeDtypeStruct((256, 256), jnp.float32),
    )(sums2d, t3, s3)


def kernel(volume, step_length, idxs, target, source):
    vol_flat = volume.reshape(-1)
    sums = _sc_sums(vol_flat, idxs.reshape(-1), step_length.reshape(-1))
    t3 = target.T.reshape(3, 256, 256)
    s3 = source.T.reshape(3, 256, 256)
    out = _tc_finish(sums.reshape(256, 256), t3, s3)
    return out.reshape(1, 1, 256, 256)


# speedup vs baseline: 1.0189x; 1.0189x over previous
import functools

import jax
import jax.numpy as jnp
from jax import lax
from jax.experimental import pallas as pl
from jax.experimental.pallas import tpu as pltpu
from jax.experimental.pallas import tpu_sc as plsc

NC = 2
NS = 16
NW = NC * NS
RAYS = 65536
SAMPLES = 256
RPW = RAYS // NW
CR = 16
CHUNKS = RPW // CR
NPAIRS = CHUNKS // 2
CI = CR * SAMPLES
KROWS = CI // 128
ROWS_PER_WORKER = RPW * SAMPLES // 128

_FLIP_MASK = 255 << 16


def _sc_sums(vol_flat, idxs2d, step_flat):
    mesh = plsc.VectorSubcoreMesh(core_axis_name="c", subcore_axis_name="s")

    @functools.partial(
        pl.kernel,
        mesh=mesh,
        out_type=jax.ShapeDtypeStruct((RAYS,), jnp.float32),
        scratch_types=[
            pltpu.VMEM((KROWS, 128), jnp.int32),
            pltpu.VMEM((KROWS, 128), jnp.int32),
            pltpu.VMEM((KROWS, 128), jnp.float32),
            pltpu.VMEM((KROWS, 128), jnp.float32),
            pltpu.VMEM((CI,), jnp.float32),
            pltpu.VMEM((CI,), jnp.float32),
            pltpu.VMEM((RPW,), jnp.float32),
            pltpu.SemaphoreType.DMA,
            pltpu.SemaphoreType.DMA,
            pltpu.SemaphoreType.DMA,
        ],
    )
    def k(vol_hbm, idx_hbm, step_hbm, out_hbm,
          tidx0, tidx1, gbuf0, gbuf1, step0, step1, outbuf,
          sem_in, sem_g0, sem_g1):
        wid = lax.axis_index("s") * NC + lax.axis_index("c")
        ray0 = wid * RPW
        row0 = wid * ROWS_PER_WORKER
        lanes = lax.iota(jnp.int32, 16)
        perms = [lanes ^ st for st in (1, 2, 4, 8)]

        def start_in(c, tidx, stepb):
            pltpu.async_copy(
                idx_hbm.at[pl.ds(row0 + c * KROWS, KROWS), :], tidx, sem_in)
            pltpu.async_copy(
                step_hbm.at[pl.ds((ray0 + c * CR) * SAMPLES, CI)], stepb,
                sem_in)

        def wait_in(c, tidx, stepb):
            pltpu.make_async_copy(
                idx_hbm.at[pl.ds(row0 + c * KROWS, KROWS), :], tidx,
                sem_in).wait()
            pltpu.make_async_copy(
                step_hbm.at[pl.ds((ray0 + c * CR) * SAMPLES, CI)], stepb,
                sem_in).wait()

        def transform(tidx):
            def trow(rw, _):
                for j in range(8):
                    sl = tidx[rw, pl.ds(j * 16, 16)]
                    tidx[rw, pl.ds(j * 16, 16)] = sl ^ _FLIP_MASK
                return 0

            lax.fori_loop(0, KROWS, trow, 0)

        def fire_g(tidx, gbuf, sem):
            def fire(jr, _):
                pltpu.async_copy(vol_hbm.at[tidx.at[jr]], gbuf.at[jr], sem)
                return 0

            lax.fori_loop(0, KROWS, fire, 0)

        def drain_g(tidx, gbuf, sem):
            def drain(jr, _):
                pltpu.make_async_copy(
                    vol_hbm.at[tidx.at[jr]], gbuf.at[jr], sem).wait()
                return 0

            lax.fori_loop(0, KROWS, drain, 0)

        def compute(c, gbuf, stepb):
            for grp in range(CR // 16):
                def ray_body(r, outvec, grp=grp):
                    rr = grp * 16 + r
                    acc = jnp.zeros((16,), jnp.float32)
                    for j in range(16):
                        row = 2 * rr + (1 if j >= 8 else 0)
                        col = (j % 8) * 16
                        g = gbuf[row, pl.ds(col, 16)]
                        s = stepb[pl.ds(rr * SAMPLES + j * 16, 16)]
                        acc = acc + g * s
                    for p in perms:
                        acc = acc + acc.at[p].get(mode="promise_in_bounds")
                    return jnp.where(lanes == r, acc, outvec)

                outvec = lax.fori_loop(
                    0, 16, ray_body, jnp.zeros((16,), jnp.float32))
                outbuf[pl.ds(c * CR + grp * 16, 16)] = outvec

        start_in(0, tidx0, step0)

        def pair_body(p, _):
            c0 = 2 * p
            c1 = c0 + 1
            wait_in(c0, tidx0, step0)
            transform(tidx0)
            fire_g(tidx0, gbuf0, sem_g0)

            @pl.when(p > 0)
            def _():
                drain_g(tidx1, gbuf1, sem_g1)
                compute(c0 - 1, gbuf1, step1)

            start_in(c1, tidx1, step1)
            wait_in(c1, tidx1, step1)
            transform(tidx1)
            fire_g(tidx1, gbuf1, sem_g1)
            drain_g(tidx0, gbuf0, sem_g0)
            compute(c0, gbuf0, step0)

            @pl.when(p < NPAIRS - 1)
            def _():
                start_in(c0 + 2, tidx0, step0)

            return 0

        lax.fori_loop(0, NPAIRS, pair_body, 0)
        drain_g(tidx1, gbuf1, sem_g1)
        compute(CHUNKS - 1, gbuf1, step1)
        pltpu.sync_copy(outbuf, out_hbm.at[pl.ds(ray0, RPW)])

    return k(vol_flat, idxs2d, step_flat)


def _tc_finish(sums2d, t3, s3):
    def body(sum_ref, t_ref, s_ref, o_ref):
        d = t_ref[...] - s_ref[...] + jnp.float32(1e-8)
        rl = jnp.sqrt(jnp.sum(d * d, axis=0))
        o_ref[...] = sum_ref[...] * rl

    return pl.pallas_call(
        body,
        out_shape=jax.ShapeDtypeStruct((256, 256), jnp.float32),
    )(sums2d, t3, s3)


def kernel(volume, step_length, idxs, target, source):
    vol_flat = volume.reshape(-1)
    sums = _sc_sums(vol_flat, idxs.reshape(-1, 128), step_length.reshape(-1))
    t3 = target.T.reshape(3, 256, 256)
    s3 = source.T.reshape(3, 256, 256)
    out = _tc_finish(sums.reshape(256, 256), t3, s3)
    return out.reshape(1, 1, 256, 256)
